# baseline (speedup 1.0000x reference)
import functools

import jax
import jax.numpy as jnp
from jax import lax
from jax.experimental import pallas as pl
from jax.experimental.pallas import tpu as pltpu
from jax.experimental.pallas import tpu_sc as plsc

NC = 2
NS = 16
NW = NC * NS

IDXW = 128
KPER = 8
CHUNK = IDXW * KPER


def _sc_gather(table, idx2):
    n_idx_rows, _ = idx2.shape
    b_tot = n_idx_rows * IDXW
    h = table.shape[1]
    b_per_w = b_tot // NW
    n_chunks = b_per_w // CHUNK

    mesh = plsc.VectorSubcoreMesh(core_axis_name="c", subcore_axis_name="s")

    @functools.partial(
        pl.kernel,
        mesh=mesh,
        compiler_params=pltpu.CompilerParams(use_tc_tiling_on_sc=False),
        out_type=jax.ShapeDtypeStruct((b_tot, h), jnp.float32),
        scratch_types=[
            pltpu.VMEM((KPER, IDXW), jnp.int32),
            pltpu.VMEM((CHUNK, h), jnp.float32),
            pltpu.SemaphoreType.DMA,
        ],
    )
    def k(table_hbm, idx_hbm, out_hbm, idx_v, rows_v, sem):
        wid = lax.axis_index("s") * NC + lax.axis_index("c")
        base = wid * b_per_w

        @pl.loop(0, n_chunks)
        def _(c):
            off = pl.multiple_of(base + c * CHUNK, CHUNK)
            idx_row = pl.multiple_of(off // IDXW, KPER)
            pltpu.sync_copy(idx_hbm.at[pl.ds(idx_row, KPER)], idx_v)
            copies = []
            for j in range(KPER):
                copies.append(pltpu.async_copy(
                    table_hbm.at[idx_v.at[j]],
                    rows_v.at[pl.ds(j * IDXW, IDXW)],
                    sem,
                ))
            for cp in copies:
                cp.wait()
            pltpu.sync_copy(rows_v, out_hbm.at[pl.ds(off, CHUNK)])

    return k(table, idx2)


def _ln_body(g_ref, p_ref, gam_ref, bet_ref, o_ref):
    reps = g_ref.shape[0] // p_ref.shape[0]
    x = g_ref[...].reshape(reps, p_ref.shape[0], p_ref.shape[1])
    x = x + p_ref[...][None]
    m = jnp.mean(x, axis=-1, keepdims=True)
    d = x - m
    v = jnp.mean(d * d, axis=-1, keepdims=True)
    o = d * lax.rsqrt(v + 1e-12) * gam_ref[...] + bet_ref[...]
    o_ref[...] = o.reshape(o_ref.shape)


def _tc_ln(gathered2, pos2, gamma2, beta2, s):
    n, h = gathered2.shape
    reps = 16
    blk = reps * s
    grid = (n // blk,)
    return pl.pallas_call(
        _ln_body,
        grid=grid,
        in_specs=[
            pl.BlockSpec((blk, h), lambda i: (i, 0)),
            pl.BlockSpec((s, h), lambda i: (0, 0)),
            pl.BlockSpec((1, h), lambda i: (0, 0)),
            pl.BlockSpec((1, h), lambda i: (0, 0)),
        ],
        out_specs=pl.BlockSpec((blk, h), lambda i: (i, 0)),
        out_shape=jax.ShapeDtypeStruct((n, h), jnp.float32),
    )(gathered2, pos2, gamma2, beta2)


def kernel(input_ids, entry_emb, pos_emb, ln_gamma, ln_beta):
    b, s = input_ids.shape
    h = entry_emb.shape[1]
    idx2 = input_ids.reshape(-1, IDXW)
    gathered = _sc_gather(entry_emb, idx2)
    normed = _tc_ln(
        gathered,
        pos_emb[:s],
        ln_gamma.reshape(1, h),
        ln_beta.reshape(1, h),
        s,
    )
    return normed.reshape(b, s, h)


# speedup vs baseline: 1.0575x; 1.0575x over previous
import functools

import jax
import jax.numpy as jnp
from jax import lax
from jax.experimental import pallas as pl
from jax.experimental.pallas import tpu as pltpu
from jax.experimental.pallas import tpu_sc as plsc

NC = 2
NS = 16
NW = NC * NS

IDXW = 128
KPER = 8
CHUNK = IDXW * KPER


def _sc_gather(table, idx2):
    n_idx_rows, _ = idx2.shape
    b_tot = n_idx_rows * IDXW
    h = table.shape[1]
    b_per_w = b_tot // NW
    n_chunks = b_per_w // CHUNK

    mesh = plsc.VectorSubcoreMesh(core_axis_name="c", subcore_axis_name="s")

    @functools.partial(
        pl.kernel,
        mesh=mesh,
        compiler_params=pltpu.CompilerParams(use_tc_tiling_on_sc=False),
        out_type=jax.ShapeDtypeStruct((b_tot, h), jnp.float32),
        scratch_types=[
            pltpu.VMEM((KPER, IDXW), jnp.int32),
            pltpu.VMEM((CHUNK, h), jnp.float32),
            pltpu.SemaphoreType.DMA,
        ],
    )
    def k(table_hbm, idx_hbm, out_hbm, idx_v, rows_v, sem):
        wid = lax.axis_index("s") * NC + lax.axis_index("c")
        base = wid * b_per_w

        @pl.loop(0, n_chunks)
        def _(c):
            off = pl.multiple_of(base + c * CHUNK, CHUNK)
            idx_row = pl.multiple_of(off // IDXW, KPER)
            pltpu.sync_copy(idx_hbm.at[pl.ds(idx_row, KPER)], idx_v)
            copies = []
            for j in range(KPER):
                copies.append(pltpu.async_copy(
                    table_hbm.at[idx_v.at[j]],
                    rows_v.at[pl.ds(j * IDXW, IDXW)],
                    sem,
                ))
            for cp in copies:
                cp.wait()
            pltpu.sync_copy(rows_v, out_hbm.at[pl.ds(off, CHUNK)])

    return k(table, idx2)


def _ln_body(g_ref, p_ref, gam_ref, bet_ref, o_ref):
    h = gam_ref.shape[-1]
    x = g_ref[...] + p_ref[...]
    a = x[..., :h]
    b = x[..., h:]
    ma = jnp.mean(a, axis=-1, keepdims=True)
    mb = jnp.mean(b, axis=-1, keepdims=True)
    da = a - ma
    db = b - mb
    va = jnp.mean(da * da, axis=-1, keepdims=True)
    vb = jnp.mean(db * db, axis=-1, keepdims=True)
    g = gam_ref[...]
    be = bet_ref[...]
    oa = da * lax.rsqrt(va + 1e-12) * g + be
    ob = db * lax.rsqrt(vb + 1e-12) * g + be
    o_ref[...] = jnp.concatenate([oa, ob], axis=-1)


def _tc_ln(gathered3, pos3, gamma3, beta3):
    n, s2, h2 = gathered3.shape
    bb = 32
    grid = (n // bb,)
    return pl.pallas_call(
        _ln_body,
        grid=grid,
        in_specs=[
            pl.BlockSpec((bb, s2, h2), lambda i: (i, 0, 0)),
            pl.BlockSpec((1, s2, h2), lambda i: (0, 0, 0)),
            pl.BlockSpec((1, 1, h2 // 2), lambda i: (0, 0, 0)),
            pl.BlockSpec((1, 1, h2 // 2), lambda i: (0, 0, 0)),
        ],
        out_specs=pl.BlockSpec((bb, s2, h2), lambda i: (i, 0, 0)),
        out_shape=jax.ShapeDtypeStruct((n, s2, h2), jnp.float32),
    )(gathered3, pos3, gamma3, beta3)


def kernel(input_ids, entry_emb, pos_emb, ln_gamma, ln_beta):
    b, s = input_ids.shape
    h = entry_emb.shape[1]
    idx2 = input_ids.reshape(-1, IDXW)
    gathered = _sc_gather(entry_emb, idx2)
    normed = _tc_ln(
        gathered.reshape(b, s // 2, 2 * h),
        pos_emb[:s].reshape(1, s // 2, 2 * h),
        ln_gamma.reshape(1, 1, h),
        ln_beta.reshape(1, 1, h),
    )
    return normed.reshape(b, s, h)


# speedup vs baseline: 1.0978x; 1.0382x over previous
import functools

import jax
import jax.numpy as jnp
from jax import lax
from jax.experimental import pallas as pl
from jax.experimental.pallas import tpu as pltpu
from jax.experimental.pallas import tpu_sc as plsc

NC = 2
NS = 16
NW = NC * NS

IDXW = 128
KPER = 8
CHUNK = IDXW * KPER


def _sc_gather(table, idx2):
    n_idx_rows, _ = idx2.shape
    b_tot = n_idx_rows * IDXW
    h = table.shape[1]
    b_per_w = b_tot // NW
    n_chunks = b_per_w // CHUNK

    mesh = plsc.VectorSubcoreMesh(core_axis_name="c", subcore_axis_name="s")

    @functools.partial(
        pl.kernel,
        mesh=mesh,
        compiler_params=pltpu.CompilerParams(use_tc_tiling_on_sc=False),
        out_type=jax.ShapeDtypeStruct((b_tot, h), jnp.float32),
        scratch_types=[
            pltpu.VMEM((KPER, IDXW), jnp.int32),
            pltpu.VMEM((CHUNK, h), jnp.float32),
            pltpu.SemaphoreType.DMA,
        ],
    )
    def k(table_hbm, idx_hbm, out_hbm, idx_v, rows_v, sem):
        wid = lax.axis_index("s") * NC + lax.axis_index("c")
        base = wid * b_per_w

        @pl.loop(0, n_chunks)
        def _(c):
            off = pl.multiple_of(base + c * CHUNK, CHUNK)
            idx_row = pl.multiple_of(off // IDXW, KPER)
            pltpu.sync_copy(idx_hbm.at[pl.ds(idx_row, KPER)], idx_v)
            copies = []
            for j in range(KPER):
                copies.append(pltpu.async_copy(
                    table_hbm.at[idx_v.at[j]],
                    rows_v.at[pl.ds(j * IDXW, IDXW)],
                    sem,
                ))
            for cp in copies:
                cp.wait()
            pltpu.sync_copy(rows_v, out_hbm.at[pl.ds(off, CHUNK)])

    return k(table, idx2)


def _ln_body(g_ref, p_ref, gam_ref, bet_ref, o_ref):
    h = gam_ref.shape[-1]
    x = g_ref[...] + p_ref[...]
    a = x[..., :h]
    b = x[..., h:]
    ma = jnp.mean(a, axis=-1, keepdims=True)
    mb = jnp.mean(b, axis=-1, keepdims=True)
    da = a - ma
    db = b - mb
    va = jnp.mean(da * da, axis=-1, keepdims=True)
    vb = jnp.mean(db * db, axis=-1, keepdims=True)
    g = gam_ref[...]
    be = bet_ref[...]
    oa = da * lax.rsqrt(va + 1e-12) * g + be
    ob = db * lax.rsqrt(vb + 1e-12) * g + be
    o = jnp.concatenate([oa, ob], axis=-1)
    o_ref[...] = jnp.transpose(o, (1, 2, 0))


def _tc_ln(gathered3, pos3, gamma3, beta3):
    n, s2, h2 = gathered3.shape
    bb = 128
    grid = (n // bb,)
    return pl.pallas_call(
        _ln_body,
        grid=grid,
        in_specs=[
            pl.BlockSpec((bb, s2, h2), lambda i: (i, 0, 0)),
            pl.BlockSpec((1, s2, h2), lambda i: (0, 0, 0)),
            pl.BlockSpec((1, 1, h2 // 2), lambda i: (0, 0, 0)),
            pl.BlockSpec((1, 1, h2 // 2), lambda i: (0, 0, 0)),
        ],
        out_specs=pl.BlockSpec((s2, h2, bb), lambda i: (0, 0, i)),
        out_shape=jax.ShapeDtypeStruct((s2, h2, n), jnp.float32),
    )(gathered3, pos3, gamma3, beta3)


def kernel(input_ids, entry_emb, pos_emb, ln_gamma, ln_beta):
    b, s = input_ids.shape
    h = entry_emb.shape[1]
    idx2 = input_ids.reshape(-1, IDXW)
    gathered = _sc_gather(entry_emb, idx2)
    normed_t = _tc_ln(
        gathered.reshape(b, s // 2, 2 * h),
        pos_emb[:s].reshape(1, s // 2, 2 * h),
        ln_gamma.reshape(1, 1, h),
        ln_beta.reshape(1, 1, h),
    )
    return jnp.transpose(normed_t, (2, 0, 1)).reshape(b, s, h)


# speedup vs baseline: 1.4373x; 1.3092x over previous
import functools

import jax
import jax.numpy as jnp
from jax import lax
from jax.experimental import pallas as pl
from jax.experimental.pallas import tpu as pltpu
from jax.experimental.pallas import tpu_sc as plsc

NC = 2
NS = 16
NW = NC * NS

IDXW = 128
KPER = 8
CHUNK = IDXW * KPER


def _sc_gather(table, idx2, hp):
    n_idx_rows, _ = idx2.shape
    b_tot = n_idx_rows * IDXW
    h = table.shape[1]
    b_per_w = b_tot // NW
    n_chunks = b_per_w // CHUNK

    mesh = plsc.VectorSubcoreMesh(core_axis_name="c", subcore_axis_name="s")

    @functools.partial(
        pl.kernel,
        mesh=mesh,
        compiler_params=pltpu.CompilerParams(use_tc_tiling_on_sc=False),
        out_type=jax.ShapeDtypeStruct((b_tot, hp), jnp.float32),
        scratch_types=[
            pltpu.VMEM((KPER, IDXW), jnp.int32),
            pltpu.VMEM((CHUNK, h), jnp.float32),
            pltpu.SemaphoreType.DMA,
        ],
    )
    def k(table_hbm, idx_hbm, out_hbm, idx_v, rows_v, sem):
        wid = lax.axis_index("s") * NC + lax.axis_index("c")
        base = wid * b_per_w

        @pl.loop(0, n_chunks)
        def _(c):
            off = pl.multiple_of(base + c * CHUNK, CHUNK)
            idx_row = pl.multiple_of(off // IDXW, KPER)
            pltpu.sync_copy(idx_hbm.at[pl.ds(idx_row, KPER)], idx_v)
            copies = []
            for j in range(KPER):
                copies.append(pltpu.async_copy(
                    table_hbm.at[idx_v.at[j]],
                    rows_v.at[pl.ds(j * IDXW, IDXW)],
                    sem,
                ))
            for cp in copies:
                cp.wait()
            pltpu.sync_copy(rows_v,
                            out_hbm.at[pl.ds(off, CHUNK), pl.ds(0, h)])

    return k(table, idx2)


def _ln_body(g_ref, p_ref, gam_ref, bet_ref, o_ref):
    h = gam_ref.shape[-1]
    x = g_ref[...][:, :h] + p_ref[0]
    m = jnp.mean(x, axis=-1, keepdims=True)
    d = x - m
    v = jnp.mean(d * d, axis=-1, keepdims=True)
    o = d * lax.rsqrt(v + 1e-12) * gam_ref[...] + bet_ref[...]
    o_ref[...] = jnp.transpose(o, (1, 0))[None]


def _tc_ln(gathered2, pos3, gamma2, beta2, b):
    n2, hp = gathered2.shape
    s, _, h = pos3.shape
    grid = (s,)
    return pl.pallas_call(
        _ln_body,
        grid=grid,
        in_specs=[
            pl.BlockSpec((b, hp), lambda i: (i, 0)),
            pl.BlockSpec((1, 1, h), lambda i: (i, 0, 0)),
            pl.BlockSpec((1, h), lambda i: (0, 0)),
            pl.BlockSpec((1, h), lambda i: (0, 0)),
        ],
        out_specs=pl.BlockSpec((1, h, b), lambda i: (i, 0, 0)),
        out_shape=jax.ShapeDtypeStruct((s, h, b), jnp.float32),
    )(gathered2, pos3, gamma2, beta2)


def kernel(input_ids, entry_emb, pos_emb, ln_gamma, ln_beta):
    b, s = input_ids.shape
    h = entry_emb.shape[1]
    hp = 2 * h
    idx2 = input_ids.T.reshape(-1, IDXW)
    gathered = _sc_gather(entry_emb, idx2, hp)
    normed_t = _tc_ln(
        gathered,
        pos_emb[:s].reshape(s, 1, h),
        ln_gamma.reshape(1, h),
        ln_beta.reshape(1, h),
        b,
    )
    return jnp.transpose(normed_t, (2, 0, 1))


# speedup vs baseline: 1.4379x; 1.0004x over previous
import functools

import jax
import jax.numpy as jnp
from jax import lax
from jax.experimental import pallas as pl
from jax.experimental.pallas import tpu as pltpu
from jax.experimental.pallas import tpu_sc as plsc

NC = 2
NS = 16
NW = NC * NS

IDXW = 128
KPER = 8
CHUNK = IDXW * KPER


def _sc_gather(table, idx2, hp):
    n_idx_rows, _ = idx2.shape
    b_tot = n_idx_rows * IDXW
    h = table.shape[1]
    b_per_w = b_tot // NW
    n_chunks = b_per_w // CHUNK

    mesh = plsc.VectorSubcoreMesh(core_axis_name="c", subcore_axis_name="s")

    @functools.partial(
        pl.kernel,
        mesh=mesh,
        compiler_params=pltpu.CompilerParams(use_tc_tiling_on_sc=False),
        out_type=jax.ShapeDtypeStruct((b_tot, hp), jnp.float32),
        scratch_types=[
            pltpu.VMEM((KPER, IDXW), jnp.int32),
            pltpu.VMEM((CHUNK, h), jnp.float32),
            pltpu.SemaphoreType.DMA,
        ],
    )
    def k(table_hbm, idx_hbm, out_hbm, idx_v, rows_v, sem):
        wid = lax.axis_index("s") * NC + lax.axis_index("c")
        base = wid * b_per_w

        @pl.loop(0, n_chunks)
        def _(c):
            off = pl.multiple_of(base + c * CHUNK, CHUNK)
            idx_row = pl.multiple_of(off // IDXW, KPER)
            pltpu.sync_copy(idx_hbm.at[pl.ds(idx_row, KPER)], idx_v)
            copies = []
            for j in range(KPER):
                copies.append(pltpu.async_copy(
                    table_hbm.at[idx_v.at[j]],
                    rows_v.at[pl.ds(j * IDXW, IDXW)],
                    sem,
                ))
            for cp in copies:
                cp.wait()
            pltpu.sync_copy(rows_v,
                            out_hbm.at[pl.ds(off, CHUNK), pl.ds(0, h)])

    return k(table, idx2)


def _ln_body(g_ref, p_ref, gam_ref, bet_ref, o_ref):
    h = gam_ref.shape[-1]
    x = g_ref[...][:, :h] + p_ref[0]
    m = jnp.mean(x, axis=-1, keepdims=True)
    d = x - m
    v = jnp.mean(d * d, axis=-1, keepdims=True)
    o = d * lax.rsqrt(v + 1e-12) * gam_ref[...] + bet_ref[...]
    o_ref[...] = jnp.transpose(o, (1, 0))[None]


def _tc_ln(gathered2, pos3, gamma2, beta2, b):
    n2, hp = gathered2.shape
    s, _, h = pos3.shape
    grid = (s,)
    return pl.pallas_call(
        _ln_body,
        grid=grid,
        compiler_params=pltpu.CompilerParams(
            dimension_semantics=("parallel",)),
        in_specs=[
            pl.BlockSpec((b, hp), lambda i: (i, 0)),
            pl.BlockSpec((1, 1, h), lambda i: (i, 0, 0)),
            pl.BlockSpec((1, h), lambda i: (0, 0)),
            pl.BlockSpec((1, h), lambda i: (0, 0)),
        ],
        out_specs=pl.BlockSpec((1, h, b), lambda i: (i, 0, 0)),
        out_shape=jax.ShapeDtypeStruct((s, h, b), jnp.float32),
    )(gathered2, pos3, gamma2, beta2)


def kernel(input_ids, entry_emb, pos_emb, ln_gamma, ln_beta):
    b, s = input_ids.shape
    h = entry_emb.shape[1]
    hp = 2 * h
    idx2 = input_ids.T.reshape(-1, IDXW)
    gathered = _sc_gather(entry_emb, idx2, hp)
    normed_t = _tc_ln(
        gathered,
        pos_emb[:s].reshape(s, 1, h),
        ln_gamma.reshape(1, h),
        ln_beta.reshape(1, h),
        b,
    )
    return jnp.transpose(normed_t, (2, 0, 1))


# speedup vs baseline: 1.5670x; 1.0897x over previous
import functools

import jax
import jax.numpy as jnp
from jax import lax
from jax.experimental import pallas as pl
from jax.experimental.pallas import tpu as pltpu
from jax.experimental.pallas import tpu_sc as plsc

NC = 2
NS = 16
NW = NC * NS

IDXW = 128
KPER = 8
CHUNK = IDXW * KPER


def _sc_gather(table, idx2, hp):
    n_idx_rows, _ = idx2.shape
    b_tot = n_idx_rows * IDXW
    h = table.shape[1]
    b_per_w = b_tot // NW
    n_chunks = b_per_w // CHUNK

    mesh = plsc.VectorSubcoreMesh(core_axis_name="c", subcore_axis_name="s")

    @functools.partial(
        pl.kernel,
        mesh=mesh,
        compiler_params=pltpu.CompilerParams(use_tc_tiling_on_sc=False),
        out_type=jax.ShapeDtypeStruct((b_tot, hp), jnp.float32),
        scratch_types=[
            pltpu.VMEM((KPER, IDXW), jnp.int32),
            pltpu.VMEM((CHUNK, h), jnp.float32),
            pltpu.SemaphoreType.DMA,
        ],
    )
    def k(table_hbm, idx_hbm, out_hbm, idx_v, rows_v, sem):
        wid = lax.axis_index("s") * NC + lax.axis_index("c")
        base = wid * b_per_w

        @pl.loop(0, n_chunks)
        def _(c):
            off = pl.multiple_of(base + c * CHUNK, CHUNK)
            idx_row = pl.multiple_of(off // IDXW, KPER)
            pltpu.sync_copy(idx_hbm.at[pl.ds(idx_row, KPER)], idx_v)
            copies = []
            for j in range(KPER):
                copies.append(pltpu.async_copy(
                    table_hbm.at[idx_v.at[j]],
                    rows_v.at[pl.ds(j * IDXW, IDXW)],
                    sem,
                ))
            for cp in copies:
                cp.wait()
            pltpu.sync_copy(rows_v,
                            out_hbm.at[pl.ds(off, CHUNK), pl.ds(0, h)])

    return k(table, idx2)


def _ln_body(g_ref, p_ref, gamt_ref, bett_ref, o_ref):
    h = gamt_ref.shape[0]
    x = g_ref[...][:, :h] + p_ref[0]
    xt = jnp.transpose(x, (1, 0))
    m = jnp.mean(xt, axis=0, keepdims=True)
    d = xt - m
    v = jnp.mean(d * d, axis=0, keepdims=True)
    o = d * lax.rsqrt(v + 1e-12) * gamt_ref[...] + bett_ref[...]
    o_ref[...] = o[None]


def _tc_ln(gathered2, pos3, gammat, betat, b):
    n2, hp = gathered2.shape
    s, _, h = pos3.shape
    grid = (s,)
    return pl.pallas_call(
        _ln_body,
        grid=grid,
        compiler_params=pltpu.CompilerParams(
            dimension_semantics=("parallel",)),
        in_specs=[
            pl.BlockSpec((b, hp), lambda i: (i, 0)),
            pl.BlockSpec((1, 1, h), lambda i: (i, 0, 0)),
            pl.BlockSpec((h, b), lambda i: (0, 0)),
            pl.BlockSpec((h, b), lambda i: (0, 0)),
        ],
        out_specs=pl.BlockSpec((1, h, b), lambda i: (i, 0, 0)),
        out_shape=jax.ShapeDtypeStruct((s, h, b), jnp.float32),
    )(gathered2, pos3, gammat, betat)


def kernel(input_ids, entry_emb, pos_emb, ln_gamma, ln_beta):
    b, s = input_ids.shape
    h = entry_emb.shape[1]
    hp = 2 * h
    idx2 = input_ids.T.reshape(-1, IDXW)
    gathered = _sc_gather(entry_emb, idx2, hp)
    normed_t = _tc_ln(
        gathered,
        pos_emb[:s].reshape(s, 1, h),
        jnp.broadcast_to(ln_gamma.reshape(h, 1), (h, b)),
        jnp.broadcast_to(ln_beta.reshape(h, 1), (h, b)),
        b,
    )
    return jnp.transpose(normed_t, (2, 0, 1))
